# baseline (device time: 26196 ns/iter reference)
import os

import jax
import jax.numpy as jnp
from jax import lax
from jax.experimental import pallas as pl
from jax.experimental.pallas import tpu as pltpu

N_DEV = 32
PLANE = 16
S = 4

_PHASES = os.environ.get("KERNEL_PHASES", "all")
_DO_X = _PHASES in ("all", "x")
_DO_PLANE = _PHASES in ("all", "plane")
_DO_BARRIER = os.environ.get("KERNEL_BARRIER", "1") == "1"


def kernel(x, w_mat):
    m, k = x.shape
    _, n = w_mat.shape
    chunk_m = m // N_DEV
    half_m = PLANE * chunk_m
    ncol = n // S

    def body(x_ref, w_ref, out_ref, xp_ref, acc_ref, xs_ref, xr_ref, ps_ref,
             pr_ref, xs_sem, xr_sem, ps_sem, pr_sem, plane_ready):
        my = lax.axis_index("i")
        b = lax.rem(my, 2)
        yq = lax.rem(my // 2, 4)
        zq = my // 8
        xcoord = b ^ lax.rem(yq, 2)
        prank = zq * 4 + yq
        partner = my ^ 1

        def plane_dest(p2, xc):
            z2 = p2 // 4
            y2 = lax.rem(p2, 4)
            return z2 * 8 + y2 * 2 + (xc ^ lax.rem(y2, 2))

        if _DO_BARRIER:
            barrier_sem = pltpu.get_barrier_semaphore()
            pl.semaphore_signal(
                barrier_sem, inc=1,
                device_id=(partner,), device_id_type=pl.DeviceIdType.MESH,
            )
            for dj in range(1, PLANE):
                p2 = lax.rem(prank + dj, PLANE)
                pl.semaphore_signal(
                    plane_ready, inc=1,
                    device_id=(plane_dest(p2, xcoord),),
                    device_id_type=pl.DeviceIdType.MESH,
                )

        for j in range(PLANE):
            p2 = lax.rem(prank + j, PLANE)
            dl_other = plane_dest(p2, 1 - xcoord)
            dl_own = plane_dest(p2, xcoord)
            xp_ref[j * chunk_m:(j + 1) * chunk_m, :] = x_ref[
                pl.ds(dl_other * chunk_m, chunk_m), :
            ].astype(jnp.bfloat16)
            xp_ref[half_m + j * chunk_m:half_m + (j + 1) * chunk_m, :] = x_ref[
                pl.ds(dl_own * chunk_m, chunk_m), :
            ].astype(jnp.bfloat16)

        acc_ref[0:half_m, :] = jnp.dot(
            xp_ref[0:half_m, :], w_ref[:, :].astype(jnp.bfloat16),
            preferred_element_type=jnp.float32,
        )

        if _DO_BARRIER:
            pl.semaphore_wait(barrier_sem, 1)
        x_rdmas = []
        for s in range(S if _DO_X else 0):
            cs = slice(s * ncol, (s + 1) * ncol)
            xs_ref[s] = acc_ref[0:half_m, cs].astype(jnp.bfloat16)
            rdma = pltpu.make_async_remote_copy(
                src_ref=xs_ref.at[s],
                dst_ref=xr_ref.at[s],
                send_sem=xs_sem.at[s],
                recv_sem=xr_sem.at[s],
                device_id=(partner,),
                device_id_type=pl.DeviceIdType.MESH,
            )
            rdma.start()
            x_rdmas.append(rdma)

        acc_ref[half_m:m, :] = jnp.dot(
            xp_ref[half_m:m, :], w_ref[:, :].astype(jnp.bfloat16),
            preferred_element_type=jnp.float32,
        )

        if _DO_BARRIER and _DO_PLANE:
            pl.semaphore_wait(plane_ready, PLANE - 1)
        p_rdmas = []
        for s in range(S if _DO_PLANE else 0):
            cs = slice(s * ncol, (s + 1) * ncol)
            if _DO_X:
                x_rdmas[s].wait_recv()
            for dj in range(1, PLANE):
                row = half_m + dj * chunk_m
                combined = acc_ref[row:row + chunk_m, cs] + xr_ref[
                    s, dj * chunk_m:(dj + 1) * chunk_m, :
                ].astype(jnp.float32)
                ps_ref[s, dj - 1] = combined.astype(jnp.bfloat16)
                rdma = pltpu.make_async_remote_copy(
                    src_ref=ps_ref.at[s, dj - 1],
                    dst_ref=pr_ref.at[s, dj - 1],
                    send_sem=ps_sem.at[s, dj - 1],
                    recv_sem=pr_sem.at[s, dj - 1],
                    device_id=(plane_dest(lax.rem(prank + dj, PLANE), xcoord),),
                    device_id_type=pl.DeviceIdType.MESH,
                )
                rdma.start()
                p_rdmas.append(rdma)

        for s in range(S):
            cs = slice(s * ncol, (s + 1) * ncol)
            if _DO_PLANE:
                for idx in range(PLANE - 1):
                    p_rdmas[s * (PLANE - 1) + idx].wait_recv()
            if _DO_X and not _DO_PLANE:
                x_rdmas[s].wait_recv()
            out_ref[:, cs] = (
                acc_ref[half_m:half_m + chunk_m, cs]
                + xr_ref[s, 0:chunk_m, :].astype(jnp.float32)
                + jnp.sum(pr_ref[s].astype(jnp.float32), axis=0)
            )

        for r in x_rdmas:
            r.wait_send()
        for r in p_rdmas:
            r.wait_send()

    return pl.pallas_call(
        body,
        out_shape=jax.ShapeDtypeStruct((chunk_m, n), jnp.float32),
        in_specs=[
            pl.BlockSpec(memory_space=pltpu.VMEM),
            pl.BlockSpec(memory_space=pltpu.VMEM),
        ],
        out_specs=pl.BlockSpec(memory_space=pltpu.VMEM),
        scratch_shapes=[
            pltpu.VMEM((m, k), jnp.bfloat16),
            pltpu.VMEM((m, n), jnp.float32),
            pltpu.VMEM((S, half_m, ncol), jnp.bfloat16),
            pltpu.VMEM((S, half_m, ncol), jnp.bfloat16),
            pltpu.VMEM((S, PLANE - 1, chunk_m, ncol), jnp.bfloat16),
            pltpu.VMEM((S, PLANE - 1, chunk_m, ncol), jnp.bfloat16),
            pltpu.SemaphoreType.DMA((S,)),
            pltpu.SemaphoreType.DMA((S,)),
            pltpu.SemaphoreType.DMA((S, PLANE - 1)),
            pltpu.SemaphoreType.DMA((S, PLANE - 1)),
            pltpu.SemaphoreType.REGULAR,
        ],
        compiler_params=pltpu.CompilerParams(
            collective_id=0 if _DO_BARRIER else None
        ),
    )(x, w_mat)


# device time: 25054 ns/iter; 1.0456x vs baseline; 1.0456x over previous
import os

import jax
import jax.numpy as jnp
from jax import lax
from jax.experimental import pallas as pl
from jax.experimental.pallas import tpu as pltpu

N_DEV = 32
PLANE = 16
S = 4

_PHASES = os.environ.get("KERNEL_PHASES", "all")
_DO_X = _PHASES in ("all", "x")
_DO_PLANE = _PHASES in ("all", "plane")
_DO_BARRIER = os.environ.get("KERNEL_BARRIER", "1") == "1"
_SCOPES = os.environ.get("KERNEL_SCOPES", "0") == "1"

import contextlib


def _scope(name):
    return jax.named_scope(name) if _SCOPES else contextlib.nullcontext()


def kernel(x, w_mat):
    m, k = x.shape
    _, n = w_mat.shape
    chunk_m = m // N_DEV
    half_m = PLANE * chunk_m
    ncol = n // S

    def body(x_ref, w_ref, out_ref, xp_ref, acc_ref, xs_ref, xr_ref, ps_ref,
             pr_ref, xs_sem, xr_sem, ps_sem, pr_sem, plane_ready):
        my = lax.axis_index("i")
        b = lax.rem(my, 2)
        yq = lax.rem(my // 2, 4)
        zq = my // 8
        xcoord = b ^ lax.rem(yq, 2)
        prank = zq * 4 + yq
        partner = my ^ 1

        def plane_dest(p2, xc):
            z2 = p2 // 4
            y2 = lax.rem(p2, 4)
            return z2 * 8 + y2 * 2 + (xc ^ lax.rem(y2, 2))

        if _DO_BARRIER:
            barrier_sem = pltpu.get_barrier_semaphore()
            pl.semaphore_signal(
                barrier_sem, inc=1,
                device_id=(partner,), device_id_type=pl.DeviceIdType.MESH,
            )
            for dj in range(1, PLANE):
                p2 = lax.rem(prank + dj, PLANE)
                pl.semaphore_signal(
                    plane_ready, inc=1,
                    device_id=(plane_dest(p2, xcoord),),
                    device_id_type=pl.DeviceIdType.MESH,
                )

        for j in range(PLANE):
            p2 = lax.rem(prank + j, PLANE)
            dl_other = plane_dest(p2, 1 - xcoord)
            dl_own = plane_dest(p2, xcoord)
            xp_ref[j * chunk_m:(j + 1) * chunk_m, :] = x_ref[
                pl.ds(dl_other * chunk_m, chunk_m), :
            ].astype(jnp.bfloat16)
            xp_ref[half_m + j * chunk_m:half_m + (j + 1) * chunk_m, :] = x_ref[
                pl.ds(dl_own * chunk_m, chunk_m), :
            ].astype(jnp.bfloat16)

        with _scope("mm1"):
            xs_ref[:, :] = jnp.dot(
                xp_ref[0:half_m, :], w_ref[:, :].astype(jnp.bfloat16),
                preferred_element_type=jnp.float32,
            ).astype(jnp.bfloat16)

        if _DO_BARRIER:
            with _scope("pwait"):
                pl.semaphore_wait(barrier_sem, 1)
        x_rdmas = []
        with _scope("xfire"):
            for s in range(S if _DO_X else 0):
                cs = slice(s * ncol, (s + 1) * ncol)
                rdma = pltpu.make_async_remote_copy(
                    src_ref=xs_ref.at[:, cs],
                    dst_ref=xr_ref.at[s],
                    send_sem=xs_sem.at[s],
                    recv_sem=xr_sem.at[s],
                    device_id=(partner,),
                    device_id_type=pl.DeviceIdType.MESH,
                )
                rdma.start()
                x_rdmas.append(rdma)

        with _scope("mm2"):
            acc_ref[0:half_m, :] = jnp.dot(
                xp_ref[half_m:m, :], w_ref[:, :].astype(jnp.bfloat16),
                preferred_element_type=jnp.float32,
            )

        if _DO_BARRIER and _DO_PLANE:
            with _scope("prwait"):
                pl.semaphore_wait(plane_ready, PLANE - 1)
        p_rdmas = []
        for s in range(S if _DO_PLANE else 0):
            cs = slice(s * ncol, (s + 1) * ncol)
            if _DO_X:
                with _scope(f"xrecv#s={s}"):
                    x_rdmas[s].wait_recv()
            for dj in range(1, PLANE):
                row = dj * chunk_m
                combined = acc_ref[row:row + chunk_m, cs] + xr_ref[
                    s, dj * chunk_m:(dj + 1) * chunk_m, :
                ].astype(jnp.float32)
                ps_ref[s, dj - 1] = combined.astype(jnp.bfloat16)
                rdma = pltpu.make_async_remote_copy(
                    src_ref=ps_ref.at[s, dj - 1],
                    dst_ref=pr_ref.at[s, dj - 1],
                    send_sem=ps_sem.at[s],
                    recv_sem=pr_sem.at[s],
                    device_id=(plane_dest(lax.rem(prank + dj, PLANE), xcoord),),
                    device_id_type=pl.DeviceIdType.MESH,
                )
                rdma.start()
                p_rdmas.append(rdma)

        for s in range(S):
            cs = slice(s * ncol, (s + 1) * ncol)
            if _DO_PLANE:
                with _scope(f"precv#s={s}"):
                    for idx in range(PLANE - 1):
                        p_rdmas[s * (PLANE - 1) + idx].wait_recv()
            if _DO_X and not _DO_PLANE:
                x_rdmas[s].wait_recv()
            with _scope(f"reduce#s={s}"):
                out_ref[:, cs] = (
                    acc_ref[0:chunk_m, cs]
                    + xr_ref[s, 0:chunk_m, :].astype(jnp.float32)
                    + jnp.sum(pr_ref[s].astype(jnp.float32), axis=0)
                )

        for r in x_rdmas:
            r.wait_send()
        for r in p_rdmas:
            r.wait_send()

    return pl.pallas_call(
        body,
        out_shape=jax.ShapeDtypeStruct((chunk_m, n), jnp.float32),
        in_specs=[
            pl.BlockSpec(memory_space=pltpu.VMEM),
            pl.BlockSpec(memory_space=pltpu.VMEM),
        ],
        out_specs=pl.BlockSpec(memory_space=pltpu.VMEM),
        scratch_shapes=[
            pltpu.VMEM((m, k), jnp.bfloat16),
            pltpu.VMEM((half_m, n), jnp.float32),
            pltpu.VMEM((half_m, n), jnp.bfloat16),
            pltpu.VMEM((S, half_m, ncol), jnp.bfloat16),
            pltpu.VMEM((S, PLANE - 1, chunk_m, ncol), jnp.bfloat16),
            pltpu.VMEM((S, PLANE - 1, chunk_m, ncol), jnp.bfloat16),
            pltpu.SemaphoreType.DMA((S,)),
            pltpu.SemaphoreType.DMA((S,)),
            pltpu.SemaphoreType.DMA((S,)),
            pltpu.SemaphoreType.DMA((S,)),
            pltpu.SemaphoreType.REGULAR,
        ],
        compiler_params=pltpu.CompilerParams(
            collective_id=0 if _DO_BARRIER else None
        ),
    )(x, w_mat)
